# Initial kernel scaffold; baseline (speedup 1.0000x reference)
import jax
import jax.numpy as jnp
from jax.experimental import pallas as pl


def kernel(edge_index, edge_label_index, edge_ones, author_ones, num_papers, num_authors):
    raise NotImplementedError("write your pallas kernel here")


# speedup vs baseline: 24.2777x; 24.2777x over previous
import jax
import jax.numpy as jnp
from jax import lax
from jax.experimental import pallas as pl
from jax.experimental.pallas import tpu as pltpu
from jax.experimental.pallas import tpu_sc as plsc

N_PAPERS = 100000
N_EDGES = 6400000
N_LABEL = 100000

NC = 2
NS = 16
NW = NC * NS

HIST_PAD = 102400
SLICE_SC = HIST_PAD // NS

BATCH = 128
ROWS_PER_CHUNK = 16
EPT = 200704
N_BATCH = EPT // BATCH
N_CHUNK = N_BATCH // ROWS_PER_CHUNK
EDGES_PAD = EPT * NW

LPT = HIST_PAD // NW


def _hist_body(dst_hbm, lbl_hbm, out_hbm, idx_v, ones_v, z_v, lbl_v, g_v, sem, hist_sh):
    c = lax.axis_index("c")
    s = lax.axis_index("s")
    wid = s * NC + c

    ones16 = jnp.ones((16,), jnp.float32)
    zeros16 = jnp.zeros((16,), jnp.float32)
    for i in range(BATCH // 16):
        ones_v[pl.ds(i * 16, 16)] = ones16

    def _zero(i, _):
        z_v[pl.ds(pl.multiple_of(i * 16, 16), 16)] = zeros16
        return _

    lax.fori_loop(0, SLICE_SC // 16, _zero, 0)
    sl = pl.ds(pl.multiple_of(s * SLICE_SC, 128), SLICE_SC)
    pltpu.sync_copy(z_v, hist_sh.at[sl])
    plsc.subcore_barrier()

    row0 = wid * N_BATCH

    def _chunk(ch, _):
        pltpu.sync_copy(
            dst_hbm.at[pl.ds(pl.multiple_of(row0 + ch * ROWS_PER_CHUNK, ROWS_PER_CHUNK), ROWS_PER_CHUNK)],
            idx_v,
        )
        for j in range(ROWS_PER_CHUNK):
            pltpu.sync_copy(ones_v, hist_sh.at[idx_v.at[j]], add=True)
        return _

    lax.fori_loop(0, N_CHUNK, _chunk, 0)
    plsc.subcore_barrier()

    pltpu.sync_copy(lbl_hbm.at[sl], lbl_v)
    pltpu.async_copy(hist_sh.at[lbl_v], g_v, sem).wait()

    @pl.when(c == 0)
    def _():
        pltpu.sync_copy(g_v, out_hbm.at[0].at[sl])

    @pl.when(c == 1)
    def _():
        pltpu.sync_copy(g_v, out_hbm.at[1].at[sl])


def _add_body(g_hbm, out_hbm, a_v, b_v):
    c = lax.axis_index("c")
    s = lax.axis_index("s")
    wid = s * NC + c
    base = pl.multiple_of(wid * LPT, 128)
    sl = pl.ds(base, LPT)
    pltpu.sync_copy(g_hbm.at[0].at[sl], a_v)
    pltpu.sync_copy(g_hbm.at[1].at[sl], b_v)

    def _add(i, _):
        o = pl.ds(pl.multiple_of(i * 16, 16), 16)
        a_v[o] = a_v[o] + b_v[o]
        return _

    lax.fori_loop(0, LPT // 16, _add, 0)
    pltpu.sync_copy(a_v, out_hbm.at[sl])


def kernel(edge_index, edge_label_index, edge_ones, author_ones, num_papers, num_authors):
    dst = edge_index[1].astype(jnp.int32)
    pad = 100000 + (jnp.arange(EDGES_PAD - N_EDGES, dtype=jnp.int32) % 352)
    dst_rows = jnp.concatenate([dst, pad]).reshape(EDGES_PAD // BATCH, BATCH)

    lbl = edge_label_index[1].astype(jnp.int32)
    lbl_pad = jnp.concatenate(
        [lbl, jnp.arange(HIST_PAD - N_LABEL, dtype=jnp.int32) % 352]
    )

    mesh = plsc.VectorSubcoreMesh(core_axis_name="c", subcore_axis_name="s")

    g = pl.kernel(
        _hist_body,
        out_type=jax.ShapeDtypeStruct((NC, HIST_PAD), jnp.float32),
        mesh=mesh,
        scratch_types=[
            pltpu.VMEM((ROWS_PER_CHUNK, BATCH), jnp.int32),
            pltpu.VMEM((BATCH,), jnp.float32),
            pltpu.VMEM((SLICE_SC,), jnp.float32),
            pltpu.VMEM((SLICE_SC,), jnp.int32),
            pltpu.VMEM((SLICE_SC,), jnp.float32),
            pltpu.SemaphoreType.DMA,
            pltpu.VMEM_SHARED((HIST_PAD,), jnp.float32),
        ],
    )(dst_rows, lbl_pad)

    out = pl.kernel(
        _add_body,
        out_type=jax.ShapeDtypeStruct((HIST_PAD,), jnp.float32),
        mesh=mesh,
        scratch_types=[
            pltpu.VMEM((LPT,), jnp.float32),
            pltpu.VMEM((LPT,), jnp.float32),
        ],
    )(g)

    return out[:N_LABEL]


# speedup vs baseline: 25.8311x; 1.0640x over previous
import jax
import jax.numpy as jnp
from jax import lax
from jax.experimental import pallas as pl
from jax.experimental.pallas import tpu as pltpu
from jax.experimental.pallas import tpu_sc as plsc

N_PAPERS = 100000
N_EDGES = 6400000
N_LABEL = 100000

NC = 2
NS = 16
NW = NC * NS

HIST_PAD = 102400
SLICE_SC = HIST_PAD // NS

BATCH = 128
ROWS_PER_CHUNK = 16
N_ROWS = N_EDGES // BATCH
ROWS_PER_TILE = 1568

LPT = HIST_PAD // NW


def _hist_body(dst_hbm, lbl_hbm, out_hbm, idx_v, ones_v, z_v, lbl_v, g_v, sem, hist_sh):
    c = lax.axis_index("c")
    s = lax.axis_index("s")
    wid = s * NC + c

    ones16 = jnp.ones((16,), jnp.float32)
    zeros16 = jnp.zeros((16,), jnp.float32)
    for i in range(BATCH // 16):
        ones_v[pl.ds(i * 16, 16)] = ones16

    def _zero(i, _):
        z_v[pl.ds(pl.multiple_of(i * 16, 16), 16)] = zeros16
        return _

    lax.fori_loop(0, SLICE_SC // 16, _zero, 0)
    sl = pl.ds(pl.multiple_of(s * SLICE_SC, 128), SLICE_SC)
    pltpu.sync_copy(z_v, hist_sh.at[sl])
    plsc.subcore_barrier()

    row0 = wid * ROWS_PER_TILE
    n_chunks = jnp.where(wid == NW - 1, 87, 98)

    def _chunk(ch, _):
        pltpu.sync_copy(
            dst_hbm.at[pl.ds(pl.multiple_of(row0 + ch * ROWS_PER_CHUNK, ROWS_PER_CHUNK), ROWS_PER_CHUNK)],
            idx_v,
        )
        for j in range(ROWS_PER_CHUNK):
            pltpu.sync_copy(ones_v, hist_sh.at[idx_v.at[j]], add=True)
        return _

    lax.fori_loop(0, n_chunks, _chunk, 0)
    plsc.subcore_barrier()

    pltpu.sync_copy(lbl_hbm.at[sl], lbl_v)
    pltpu.async_copy(hist_sh.at[lbl_v], g_v, sem).wait()

    @pl.when(c == 0)
    def _():
        pltpu.sync_copy(g_v, out_hbm.at[0].at[sl])

    @pl.when(c == 1)
    def _():
        pltpu.sync_copy(g_v, out_hbm.at[1].at[sl])


def _add_body(g_hbm, out_hbm, a_v, b_v):
    c = lax.axis_index("c")
    s = lax.axis_index("s")
    wid = s * NC + c
    base = pl.multiple_of(wid * LPT, 128)
    sl = pl.ds(base, LPT)
    pltpu.sync_copy(g_hbm.at[0].at[sl], a_v)
    pltpu.sync_copy(g_hbm.at[1].at[sl], b_v)

    def _add(i, _):
        o = pl.ds(pl.multiple_of(i * 16, 16), 16)
        a_v[o] = a_v[o] + b_v[o]
        return _

    lax.fori_loop(0, LPT // 16, _add, 0)
    pltpu.sync_copy(a_v, out_hbm.at[sl])


def kernel(edge_index, edge_label_index, edge_ones, author_ones, num_papers, num_authors):
    dst_rows = edge_index[1].astype(jnp.int32).reshape(N_ROWS, BATCH)

    lbl = edge_label_index[1].astype(jnp.int32)
    lbl_pad = jnp.concatenate(
        [lbl, jnp.arange(HIST_PAD - N_LABEL, dtype=jnp.int32) % 352]
    )

    mesh = plsc.VectorSubcoreMesh(core_axis_name="c", subcore_axis_name="s")

    g = pl.kernel(
        _hist_body,
        out_type=jax.ShapeDtypeStruct((NC, HIST_PAD), jnp.float32),
        mesh=mesh,
        scratch_types=[
            pltpu.VMEM((ROWS_PER_CHUNK, BATCH), jnp.int32),
            pltpu.VMEM((BATCH,), jnp.float32),
            pltpu.VMEM((SLICE_SC,), jnp.float32),
            pltpu.VMEM((SLICE_SC,), jnp.int32),
            pltpu.VMEM((SLICE_SC,), jnp.float32),
            pltpu.SemaphoreType.DMA,
            pltpu.VMEM_SHARED((HIST_PAD,), jnp.float32),
        ],
    )(dst_rows, lbl_pad)

    out = pl.kernel(
        _add_body,
        out_type=jax.ShapeDtypeStruct((HIST_PAD,), jnp.float32),
        mesh=mesh,
        scratch_types=[
            pltpu.VMEM((LPT,), jnp.float32),
            pltpu.VMEM((LPT,), jnp.float32),
        ],
    )(g)

    return out[:N_LABEL]
